# baseline (speedup 1.0000x reference)
import functools

import numpy as np
import jax
import jax.numpy as jnp
from jax import lax
from jax.experimental import pallas as pl
from jax.experimental.pallas import tpu as pltpu
from jax.experimental.pallas import tpu_sc as plsc

VOCAB = 100000
D_MODEL = 768
BATCH = 4
SEQ = 2048
SCALE = float(np.sqrt(float(D_MODEL)))

NC = 2
NS = 16
NW = NC * NS
PB = SEQ // NW
C = 32
NCH = PB // C
LG = D_MODEL // 16


def _positional_encoding_np(length, depth):
    half_depth = depth // 2
    positions = np.arange(length)[:, np.newaxis]
    exponents = np.arange(half_depth)[np.newaxis, :] * 2 / depth
    denom = 10000 ** exponents
    angles = positions / denom
    pos_encoding = np.zeros((length, depth), dtype=np.float64)
    pos_encoding[:, ::2] = np.sin(angles)
    pos_encoding[:, 1::2] = np.cos(angles)
    return pos_encoding.astype(np.float32)


_MESH = plsc.VectorSubcoreMesh(core_axis_name="c", subcore_axis_name="s")


@functools.partial(
    pl.kernel,
    mesh=_MESH,
    out_type=jax.ShapeDtypeStruct((BATCH * SEQ, D_MODEL), jnp.float32),
    scratch_types=[
        pltpu.VMEM((BATCH, NCH, C), jnp.int32),
        pltpu.VMEM((PB, D_MODEL), jnp.float32),
        pltpu.VMEM((C, D_MODEL), jnp.float32),
        pltpu.SemaphoreType.DMA,
    ],
)
def _embed(table_hbm, idx_hbm, pos_hbm, out_hbm, idx_v, pos_v, emb_v, sem):
    wid = lax.axis_index("s") * NC + lax.axis_index("c")
    pbase = wid * PB

    pltpu.sync_copy(idx_hbm.at[wid], idx_v)
    pltpu.sync_copy(pos_hbm.at[pl.ds(pbase, PB)], pos_v)

    def do_chunk(b, c):
        pltpu.async_copy(table_hbm.at[idx_v.at[b, c]], emb_v, sem).wait()

        def row_body(r, _):
            def col_body(j, _):
                sl = pl.ds(j * 16, 16)
                emb_v[r, sl] = emb_v[r, sl] * SCALE + pos_v[c * C + r, sl]
                return 0
            lax.fori_loop(0, LG, col_body, 0)
            return 0
        lax.fori_loop(0, C, row_body, 0)

        out_base = b * SEQ + pbase + c * C
        pltpu.sync_copy(emb_v, out_hbm.at[pl.ds(out_base, C)])

    for b in range(BATCH):
        for c in range(NCH):
            do_chunk(b, c)


def kernel(x, table):
    pos = jnp.asarray(_positional_encoding_np(SEQ, D_MODEL))
    idx = (
        x.astype(jnp.int32)
        .reshape(BATCH, NW, NCH * C)
        .transpose(1, 0, 2)
        .reshape(NW, BATCH, NCH, C)
    )
    out = _embed(table, idx, pos)
    return out.reshape(BATCH, SEQ, D_MODEL)


# speedup vs baseline: 1.6130x; 1.6130x over previous
import functools

import numpy as np
import jax
import jax.numpy as jnp
from jax import lax
from jax.experimental import pallas as pl
from jax.experimental.pallas import tpu as pltpu
from jax.experimental.pallas import tpu_sc as plsc

VOCAB = 100000
D_MODEL = 768
BATCH = 4
SEQ = 2048
SCALE = float(np.sqrt(float(D_MODEL)))

NC = 2
NS = 16
NW = NC * NS
PB = SEQ // NW
C = 32
NCH = PB // C
LG = D_MODEL // 16


def _positional_encoding_np(length, depth):
    half_depth = depth // 2
    positions = np.arange(length)[:, np.newaxis]
    exponents = np.arange(half_depth)[np.newaxis, :] * 2 / depth
    denom = 10000 ** exponents
    angles = positions / denom
    pos_encoding = np.zeros((length, depth), dtype=np.float64)
    pos_encoding[:, ::2] = np.sin(angles)
    pos_encoding[:, 1::2] = np.cos(angles)
    return pos_encoding.astype(np.float32)


_MESH = plsc.VectorSubcoreMesh(core_axis_name="c", subcore_axis_name="s")


NCHT = BATCH * NCH


@functools.partial(
    pl.kernel,
    mesh=_MESH,
    out_type=jax.ShapeDtypeStruct((BATCH * SEQ, D_MODEL), jnp.float32),
    scratch_types=[
        pltpu.VMEM((BATCH, NCH, C), jnp.int32),
        pltpu.VMEM((PB, D_MODEL), jnp.float32),
        pltpu.VMEM((C, D_MODEL), jnp.float32),
        pltpu.VMEM((C, D_MODEL), jnp.float32),
        pltpu.SemaphoreType.DMA,
        pltpu.SemaphoreType.DMA,
        pltpu.SemaphoreType.DMA,
        pltpu.SemaphoreType.DMA,
    ],
)
def _embed(table_hbm, idx_hbm, pos_hbm, out_hbm,
           idx_v, pos_v, emb0_v, emb1_v, g0, g1, w0, w1):
    wid = lax.axis_index("s") * NC + lax.axis_index("c")
    pbase = wid * PB
    bufs = (emb0_v, emb1_v)
    gsems = (g0, g1)
    wsems = (w0, w1)

    pltpu.sync_copy(idx_hbm.at[wid], idx_v)
    pltpu.sync_copy(pos_hbm.at[pl.ds(pbase, PB)], pos_v)

    def start_gather(k, bi):
        b, c = divmod(k, NCH)
        return pltpu.async_copy(table_hbm.at[idx_v.at[b, c]], bufs[bi],
                                gsems[bi])

    def start_write(k, bi):
        b, c = divmod(k, NCH)
        out_base = b * SEQ + pbase + c * C
        return pltpu.async_copy(bufs[bi], out_hbm.at[pl.ds(out_base, C)],
                                wsems[bi])

    gcopy = [None, None]
    wcopy = [None, None]
    gcopy[0] = start_gather(0, 0)
    for k in range(NCHT):
        bi = k & 1
        gcopy[bi].wait()
        if k + 1 < NCHT:
            ni = (k + 1) & 1
            if wcopy[ni] is not None:
                wcopy[ni].wait()
            gcopy[ni] = start_gather(k + 1, ni)

        buf = bufs[bi]
        poff = (k % NCH) * C

        def row_body(r, _):
            for j in range(LG):
                sl = pl.ds(j * 16, 16)
                buf[r, sl] = buf[r, sl] * SCALE + pos_v[poff + r, sl]
            return 0
        lax.fori_loop(0, C, row_body, 0)

        wcopy[bi] = start_write(k, bi)

    wcopy[0].wait()
    wcopy[1].wait()


def kernel(x, table):
    pos = jnp.asarray(_positional_encoding_np(SEQ, D_MODEL))
    idx = (
        x.astype(jnp.int32)
        .reshape(BATCH, NW, NCH * C)
        .transpose(1, 0, 2)
        .reshape(NW, BATCH, NCH, C)
    )
    out = _embed(table, idx, pos)
    return out.reshape(BATCH, SEQ, D_MODEL)


# speedup vs baseline: 2.0790x; 1.2889x over previous
import functools

import numpy as np
import jax
import jax.numpy as jnp
from jax import lax
from jax.experimental import pallas as pl
from jax.experimental.pallas import tpu as pltpu
from jax.experimental.pallas import tpu_sc as plsc

VOCAB = 100000
D_MODEL = 768
BATCH = 4
SEQ = 2048
SCALE = float(np.sqrt(float(D_MODEL)))

NC = 2
NS = 16
NW = NC * NS
PB = SEQ // NW
C = 32
NCH = PB // C
LG = D_MODEL // 16


def _positional_encoding_np(length, depth):
    half_depth = depth // 2
    positions = np.arange(length)[:, np.newaxis]
    exponents = np.arange(half_depth)[np.newaxis, :] * 2 / depth
    denom = 10000 ** exponents
    angles = positions / denom
    pos_encoding = np.zeros((length, depth), dtype=np.float64)
    pos_encoding[:, ::2] = np.sin(angles)
    pos_encoding[:, 1::2] = np.cos(angles)
    return pos_encoding.astype(np.float32)


_MESH = plsc.VectorSubcoreMesh(core_axis_name="c", subcore_axis_name="s")


NCHT = BATCH * NCH


@functools.partial(
    pl.kernel,
    mesh=_MESH,
    out_type=jax.ShapeDtypeStruct((BATCH * SEQ, D_MODEL), jnp.float32),
    scratch_types=[
        pltpu.VMEM((BATCH, NCH, C), jnp.int32),
        pltpu.VMEM((PB, D_MODEL), jnp.float32),
        pltpu.VMEM((C, D_MODEL), jnp.float32),
        pltpu.VMEM((C, D_MODEL), jnp.float32),
        pltpu.SemaphoreType.DMA,
        pltpu.SemaphoreType.DMA,
        pltpu.SemaphoreType.DMA,
        pltpu.SemaphoreType.DMA,
    ],
)
def _embed(table_hbm, idx_hbm, pos_hbm, out_hbm,
           idx_v, pos_v, emb0_v, emb1_v, g0, g1, w0, w1):
    wid = lax.axis_index("s") * NC + lax.axis_index("c")
    pbase = wid * PB
    bufs = (emb0_v, emb1_v)
    gsems = (g0, g1)
    wsems = (w0, w1)

    pltpu.sync_copy(idx_hbm.at[wid], idx_v)
    pltpu.sync_copy(pos_hbm.at[pl.ds(pbase, PB)], pos_v)

    def start_gather(k, bi):
        b, c = divmod(k, NCH)
        return pltpu.async_copy(table_hbm.at[idx_v.at[b, c]], bufs[bi],
                                gsems[bi])

    def start_write(k, bi):
        b, c = divmod(k, NCH)
        out_base = b * SEQ + pbase + c * C
        return pltpu.async_copy(bufs[bi], out_hbm.at[pl.ds(out_base, C)],
                                wsems[bi])

    gcopy = [None, None]
    wcopy = [None, None]
    gcopy[0] = start_gather(0, 0)
    for k in range(NCHT):
        bi = k & 1
        gcopy[bi].wait()
        if k + 1 < NCHT:
            ni = (k + 1) & 1
            if wcopy[ni] is not None:
                wcopy[ni].wait()
            gcopy[ni] = start_gather(k + 1, ni)

        buf = bufs[bi]
        poff = (k % NCH) * C

        @plsc.parallel_loop(0, C, 1, unroll=2)
        def row_body(r):
            for j in range(LG):
                sl = pl.ds(j * 16, 16)
                buf[r, sl] = buf[r, sl] * SCALE + pos_v[poff + r, sl]

        wcopy[bi] = start_write(k, bi)

    wcopy[0].wait()
    wcopy[1].wait()


def kernel(x, table):
    pos = jnp.asarray(_positional_encoding_np(SEQ, D_MODEL))
    idx = (
        x.astype(jnp.int32)
        .reshape(BATCH, NW, NCH * C)
        .transpose(1, 0, 2)
        .reshape(NW, BATCH, NCH, C)
    )
    out = _embed(table, idx, pos)
    return out.reshape(BATCH, SEQ, D_MODEL)
